# baseline (speedup 1.0000x reference)
import functools

import jax
import jax.numpy as jnp
from jax import lax
from jax.experimental import pallas as pl
from jax.experimental.pallas import tpu as pltpu
from jax.experimental.pallas import tpu_sc as plsc

B, N, K = 4, 10000, 16
D = 128
DW = D // 2
NC, NS = 2, 16
NW = NC * NS
CHUNK = 8
ROWS = CHUNK * K
NCHUNKS = N // CHUNK
NBUF = 5
NSLOT = -(-((NCHUNKS + NW - 1) // NW) // NBUF) * NBUF
NGRP = D // 32
BR = 2000
BN_EPS = 1e-5


def _mm_body(x_ref, w_ref, h_ref):
    h = lax.dot_general(
        x_ref[0], w_ref[...],
        dimension_numbers=(((1,), (1,)), ((), ())),
        preferred_element_type=jnp.float32,
    )
    lo = lax.bitcast_convert_type(
        h[:, :DW].astype(jnp.bfloat16), jnp.uint16).astype(jnp.uint32)
    hi = lax.bitcast_convert_type(
        h[:, DW:].astype(jnp.bfloat16), jnp.uint16).astype(jnp.uint32)
    h_ref[...] = lax.bitcast_convert_type((hi << 16) | lo, jnp.int32)


def _matmul(x, w, bidx):
    return pl.pallas_call(
        _mm_body,
        grid=(N // BR,),
        in_specs=[
            pl.BlockSpec((1, BR, D), lambda i, _b=bidx: (_b, i, 0)),
            pl.BlockSpec((D, D), lambda i: (0, 0)),
        ],
        out_specs=pl.BlockSpec((BR, DW), lambda i: (i, 0)),
        out_shape=jax.ShapeDtypeStruct((N, DW), jnp.int32),
    )(x, w)


_MESH = plsc.VectorSubcoreMesh(
    core_axis_name="c", subcore_axis_name="s", num_cores=NC, num_subcores=NS)


@functools.partial(
    pl.kernel,
    out_type=(
        jax.ShapeDtypeStruct((N, D), jnp.float32),
        jax.ShapeDtypeStruct((NW, 2 * D), jnp.float32),
    ),
    mesh=_MESH,
    compiler_params=pltpu.CompilerParams(
        use_tc_tiling_on_sc=False, needs_layout_passes=False),
    scratch_types=[
        pltpu.VMEM((NBUF, ROWS), jnp.int32),
        pltpu.VMEM((NBUF, ROWS, DW), jnp.int32),
        pltpu.VMEM((NBUF, CHUNK, D), jnp.float32),
        pltpu.VMEM((2 * D,), jnp.float32),
        [pltpu.SemaphoreType.DMA] * NBUF,
        [pltpu.SemaphoreType.DMA] * NBUF,
        [pltpu.SemaphoreType.DMA] * NBUF,
    ],
)
def _sc_gather_max(h_hbm, idx_hbm, m_hbm, part_hbm,
                   idx_v, rows_v, out_v, stat_v, isems, gsems, osems):
    cid = lax.axis_index("c")
    sid = lax.axis_index("s")
    wid = sid * NC + cid

    def chunk_base(t):
        g = jnp.minimum(t * NW + wid, NCHUNKS - 1)
        return g * CHUNK

    def issue_idx(t, b):
        pltpu.async_copy(idx_hbm.at[pl.ds(chunk_base(t) * K, ROWS)],
                         idx_v.at[b], isems[b])

    def issue_gather(t, b):
        pltpu.make_async_copy(idx_hbm.at[pl.ds(chunk_base(t) * K, ROWS)],
                              idx_v.at[b], isems[b]).wait()
        pltpu.async_copy(h_hbm.at[idx_v.at[b]], rows_v.at[b], gsems[b])

    for b in range(NBUF):
        issue_idx(b, b)
    for b in range(NBUF - 1):
        issue_gather(b, b)

    def chunk_compute(t, b, stats):
        validf = jnp.where(t * NW + wid < NCHUNKS, 1.0, 0.0).astype(jnp.float32)

        def as_bf(bits):
            return plsc.bitcast(bits, jnp.bfloat16)

        def point_body(p, carry):
            stats_in = carry
            acc = [as_bf(rows_v[b, p * K, pl.ds(g * 16, 16)])
                   for g in range(NGRP)]
            for j in range(1, K):
                for g in range(NGRP):
                    acc[g] = jnp.maximum(
                        acc[g], as_bf(rows_v[b, p * K + j, pl.ds(g * 16, 16)]))
            new_stats = list(stats_in)
            for g in range(NGRP):
                lo, hi = plsc.unpack(acc[g], format=plsc.PackFormat.INTERLEAVED)
                out_v[b, p, pl.ds(g * 16, 16)] = lo
                out_v[b, p, pl.ds(DW + g * 16, 16)] = hi
                mlo = lo * validf
                mhi = hi * validf
                new_stats[g] = stats_in[g] + mlo
                new_stats[NGRP + g] = stats_in[NGRP + g] + mhi
                new_stats[2 * NGRP + g] = stats_in[2 * NGRP + g] + mlo * lo
                new_stats[3 * NGRP + g] = stats_in[3 * NGRP + g] + mhi * hi
            return tuple(new_stats)

        return lax.fori_loop(0, CHUNK, point_body, stats)

    def outer_body(o, stats):
        for b in range(NBUF):
            t = o * NBUF + b
            bp = (b - 1) % NBUF

            @pl.when(t + NBUF - 1 < NSLOT)
            def _front_gather():
                issue_gather(t + NBUF - 1, bp)

            pltpu.make_async_copy(
                h_hbm.at[idx_v.at[b]], rows_v.at[b], gsems[b]).wait()

            @pl.when(t + NBUF < NSLOT)
            def _front_idx():
                issue_idx(t + NBUF, b)

            @pl.when(t >= NBUF)
            def _wait_out():
                pltpu.make_async_copy(
                    out_v.at[b],
                    m_hbm.at[pl.ds(chunk_base(t - NBUF), CHUNK)],
                    osems[b]).wait()

            stats = chunk_compute(t, b, stats)

            pltpu.async_copy(
                out_v.at[b], m_hbm.at[pl.ds(chunk_base(t), CHUNK)],
                osems[b])
        return stats

    zeros = tuple(jnp.zeros((16,), jnp.float32) for _ in range(4 * NGRP))
    stats = lax.fori_loop(0, NSLOT // NBUF, outer_body, zeros)

    for b in range(NBUF):
        pltpu.make_async_copy(
            out_v.at[b],
            m_hbm.at[pl.ds(chunk_base(NSLOT - NBUF + b), CHUNK)],
            osems[b]).wait()

    for g in range(NGRP):
        stat_v[pl.ds(g * 16, 16)] = stats[g]
        stat_v[pl.ds(DW + g * 16, 16)] = stats[NGRP + g]
        stat_v[pl.ds(D + g * 16, 16)] = stats[2 * NGRP + g]
        stat_v[pl.ds(D + DW + g * 16, 16)] = stats[3 * NGRP + g]
    pltpu.sync_copy(stat_v, part_hbm.at[wid])


def _bn_body(m_ref, part_ref, bnw_ref, bnb_ref, *rest):
    y_ref = rest[-1]
    part = part_ref[...]
    total = jnp.sum(part, axis=0, keepdims=True)
    mean = total[:, :D] / (B * N)
    var = total[:, D:] / (B * N) - mean * mean
    scale = bnw_ref[...] * lax.rsqrt(var + BN_EPS)
    off = bnb_ref[...] - mean * scale
    y_ref[...] = m_ref[...] * scale + off


def _batchnorm_into(mb, part, bnw, bnb, y_buf, bidx):
    in_specs = [
        pl.BlockSpec((BR, D), lambda i: (i, 0)),
        pl.BlockSpec((B * NW, 2 * D), lambda i: (0, 0)),
        pl.BlockSpec((1, D), lambda i: (0, 0)),
        pl.BlockSpec((1, D), lambda i: (0, 0)),
    ]
    args = [mb, part, bnw, bnb]
    aliases = {}
    if y_buf is not None:
        in_specs.append(pl.BlockSpec(memory_space=pl.ANY))
        args.append(y_buf)
        aliases = {4: 0}
    return pl.pallas_call(
        _bn_body,
        grid=(N // BR,),
        in_specs=in_specs,
        out_specs=pl.BlockSpec(
            (BR, D), lambda i, _b=bidx: (_b * (N // BR) + i, 0)),
        out_shape=jax.ShapeDtypeStruct((B * N, D), jnp.float32),
        input_output_aliases=aliases,
    )(*args)


def kernel(x, knn, W, bn_weight, bn_bias):
    ms, parts = [], []
    for b in range(B):
        h_b = _matmul(x, W, b)
        idx_b = knn[b].reshape(N * K)
        m_b, part_b = _sc_gather_max(h_b, idx_b)
        ms.append(m_b)
        parts.append(part_b)
    part = jnp.concatenate(parts, axis=0)
    y = None
    for b in range(B):
        y = _batchnorm_into(ms[b], part, bn_weight.reshape(1, D),
                            bn_bias.reshape(1, D), y, b)
    return y.reshape(B, N, D)


# speedup vs baseline: 1.0060x; 1.0060x over previous
import functools

import jax
import jax.numpy as jnp
from jax import lax
from jax.experimental import pallas as pl
from jax.experimental.pallas import tpu as pltpu
from jax.experimental.pallas import tpu_sc as plsc

B, N, K = 4, 10000, 16
D = 128
DW = D // 2
NC, NS = 2, 16
NW = NC * NS
CHUNK = 8
ROWS = CHUNK * K
NCHUNKS = N // CHUNK
NBUF = 4
NSLOT = -(-((NCHUNKS + NW - 1) // NW) // NBUF) * NBUF
NGRP = D // 32
BR = 2000
BN_EPS = 1e-5


def _mm_body(x_ref, w_ref, h_ref):
    h = lax.dot_general(
        x_ref[0], w_ref[...],
        dimension_numbers=(((1,), (1,)), ((), ())),
        preferred_element_type=jnp.float32,
    )
    lo = lax.bitcast_convert_type(
        h[:, :DW].astype(jnp.bfloat16), jnp.uint16).astype(jnp.uint32)
    hi = lax.bitcast_convert_type(
        h[:, DW:].astype(jnp.bfloat16), jnp.uint16).astype(jnp.uint32)
    h_ref[...] = lax.bitcast_convert_type((hi << 16) | lo, jnp.int32)


def _matmul(x, w, bidx):
    return pl.pallas_call(
        _mm_body,
        grid=(N // BR,),
        in_specs=[
            pl.BlockSpec((1, BR, D), lambda i, _b=bidx: (_b, i, 0)),
            pl.BlockSpec((D, D), lambda i: (0, 0)),
        ],
        out_specs=pl.BlockSpec((BR, DW), lambda i: (i, 0)),
        out_shape=jax.ShapeDtypeStruct((N, DW), jnp.int32),
    )(x, w)


_MESH = plsc.VectorSubcoreMesh(
    core_axis_name="c", subcore_axis_name="s", num_cores=NC, num_subcores=NS)


@functools.partial(
    pl.kernel,
    out_type=(
        jax.ShapeDtypeStruct((N, D), jnp.float32),
        jax.ShapeDtypeStruct((NW, 2 * D), jnp.float32),
    ),
    mesh=_MESH,
    compiler_params=pltpu.CompilerParams(
        use_tc_tiling_on_sc=False, needs_layout_passes=False),
    scratch_types=[
        pltpu.VMEM((NBUF, ROWS), jnp.int32),
        pltpu.VMEM((NBUF, ROWS, DW), jnp.int32),
        pltpu.VMEM((NBUF, CHUNK, D), jnp.float32),
        pltpu.VMEM((2 * D,), jnp.float32),
        [pltpu.SemaphoreType.DMA] * NBUF,
        [pltpu.SemaphoreType.DMA] * NBUF,
        [pltpu.SemaphoreType.DMA] * NBUF,
    ],
)
def _sc_gather_max(h_hbm, idx_hbm, m_hbm, part_hbm,
                   idx_v, rows_v, out_v, stat_v, isems, gsems, osems):
    cid = lax.axis_index("c")
    sid = lax.axis_index("s")
    wid = sid * NC + cid

    def chunk_base(t):
        g = jnp.minimum(t * NW + wid, NCHUNKS - 1)
        return g * CHUNK

    def issue_idx(t, b):
        pltpu.async_copy(idx_hbm.at[pl.ds(chunk_base(t) * K, ROWS)],
                         idx_v.at[b], isems[b])

    def issue_gather(t, b):
        pltpu.make_async_copy(idx_hbm.at[pl.ds(chunk_base(t) * K, ROWS)],
                              idx_v.at[b], isems[b]).wait()
        pltpu.async_copy(h_hbm.at[idx_v.at[b]], rows_v.at[b], gsems[b])

    for b in range(NBUF):
        issue_idx(b, b)
    for b in range(NBUF - 1):
        issue_gather(b, b)

    def chunk_compute(t, b, stats):
        validf = jnp.where(t * NW + wid < NCHUNKS, 1.0, 0.0).astype(jnp.float32)

        def as_bf(bits):
            return plsc.bitcast(bits, jnp.bfloat16)

        def point_body(p, carry):
            stats_in = carry
            acc = [as_bf(rows_v[b, p * K, pl.ds(g * 16, 16)])
                   for g in range(NGRP)]
            for j in range(1, K):
                for g in range(NGRP):
                    acc[g] = jnp.maximum(
                        acc[g], as_bf(rows_v[b, p * K + j, pl.ds(g * 16, 16)]))
            new_stats = list(stats_in)
            for g in range(NGRP):
                lo, hi = plsc.unpack(acc[g], format=plsc.PackFormat.INTERLEAVED)
                out_v[b, p, pl.ds(g * 16, 16)] = lo
                out_v[b, p, pl.ds(DW + g * 16, 16)] = hi
                mlo = lo * validf
                mhi = hi * validf
                new_stats[g] = stats_in[g] + mlo
                new_stats[NGRP + g] = stats_in[NGRP + g] + mhi
                new_stats[2 * NGRP + g] = stats_in[2 * NGRP + g] + mlo * lo
                new_stats[3 * NGRP + g] = stats_in[3 * NGRP + g] + mhi * hi
            return tuple(new_stats)

        return lax.fori_loop(0, CHUNK, point_body, stats)

    def outer_body(o, stats):
        for b in range(NBUF):
            t = o * NBUF + b
            bp = (b - 1) % NBUF

            @pl.when(t + NBUF - 1 < NSLOT)
            def _front_gather():
                issue_gather(t + NBUF - 1, bp)

            pltpu.make_async_copy(
                h_hbm.at[idx_v.at[b]], rows_v.at[b], gsems[b]).wait()

            @pl.when(t + NBUF < NSLOT)
            def _front_idx():
                issue_idx(t + NBUF, b)

            @pl.when(jnp.logical_and(
                t >= NBUF, (t - NBUF) * NW + wid < NCHUNKS))
            def _wait_out():
                pltpu.make_async_copy(
                    out_v.at[b],
                    m_hbm.at[pl.ds(chunk_base(t - NBUF), CHUNK)],
                    osems[b]).wait()

            stats = chunk_compute(t, b, stats)

            @pl.when(t * NW + wid < NCHUNKS)
            def _issue_out():
                pltpu.async_copy(
                    out_v.at[b], m_hbm.at[pl.ds(chunk_base(t), CHUNK)],
                    osems[b])
        return stats

    zeros = tuple(jnp.zeros((16,), jnp.float32) for _ in range(4 * NGRP))
    stats = lax.fori_loop(0, NSLOT // NBUF, outer_body, zeros)

    for b in range(NBUF):
        s = NSLOT - NBUF + b

        @pl.when(s * NW + wid < NCHUNKS)
        def _drain_out(_s=s, _b=b):
            pltpu.make_async_copy(
                out_v.at[_b],
                m_hbm.at[pl.ds(chunk_base(_s), CHUNK)],
                osems[_b]).wait()

    for g in range(NGRP):
        stat_v[pl.ds(g * 16, 16)] = stats[g]
        stat_v[pl.ds(DW + g * 16, 16)] = stats[NGRP + g]
        stat_v[pl.ds(D + g * 16, 16)] = stats[2 * NGRP + g]
        stat_v[pl.ds(D + DW + g * 16, 16)] = stats[3 * NGRP + g]
    pltpu.sync_copy(stat_v, part_hbm.at[wid])


def _bn_body(m_ref, part_ref, bnw_ref, bnb_ref, *rest):
    y_ref = rest[-1]
    part = part_ref[...]
    total = jnp.sum(part, axis=0, keepdims=True)
    mean = total[:, :D] / (B * N)
    var = total[:, D:] / (B * N) - mean * mean
    scale = bnw_ref[...] * lax.rsqrt(var + BN_EPS)
    off = bnb_ref[...] - mean * scale
    y_ref[...] = m_ref[...] * scale + off


def _batchnorm_into(mb, part, bnw, bnb, y_buf, bidx):
    in_specs = [
        pl.BlockSpec((BR, D), lambda i: (i, 0)),
        pl.BlockSpec((B * NW, 2 * D), lambda i: (0, 0)),
        pl.BlockSpec((1, D), lambda i: (0, 0)),
        pl.BlockSpec((1, D), lambda i: (0, 0)),
    ]
    args = [mb, part, bnw, bnb]
    aliases = {}
    if y_buf is not None:
        in_specs.append(pl.BlockSpec(memory_space=pl.ANY))
        args.append(y_buf)
        aliases = {4: 0}
    return pl.pallas_call(
        _bn_body,
        grid=(N // BR,),
        in_specs=in_specs,
        out_specs=pl.BlockSpec(
            (BR, D), lambda i, _b=bidx: (_b * (N // BR) + i, 0)),
        out_shape=jax.ShapeDtypeStruct((B * N, D), jnp.float32),
        input_output_aliases=aliases,
    )(*args)


def kernel(x, knn, W, bn_weight, bn_bias):
    ms, parts = [], []
    for b in range(B):
        h_b = _matmul(x, W, b)
        idx_b = knn[b].reshape(N * K)
        m_b, part_b = _sc_gather_max(h_b, idx_b)
        ms.append(m_b)
        parts.append(part_b)
    part = jnp.concatenate(parts, axis=0)
    y = None
    for b in range(B):
        y = _batchnorm_into(ms[b], part, bn_weight.reshape(1, D),
                            bn_bias.reshape(1, D), y, b)
    return y.reshape(B, N, D)
